# baseline (speedup 1.0000x reference)
import functools

import jax
import jax.numpy as jnp
from jax import lax
from jax.experimental import pallas as pl
from jax.experimental.pallas import tpu as pltpu
from jax.experimental.pallas import tpu_sc as plsc

_NC = 2
_NS = 16
_K = 128


@functools.lru_cache(maxsize=None)
def _edge_pass_kernel(NP, EP, HH):
    eper = EP // _NS
    steps = eper // _K
    rps = NP // _NS
    mesh = plsc.VectorSubcoreMesh(core_axis_name="c", subcore_axis_name="s",
                                  num_cores=_NC, num_subcores=_NS)

    @functools.partial(
        pl.kernel,
        out_type=jax.ShapeDtypeStruct((_NC, NP, HH), jnp.float32),
        mesh=mesh,
        scratch_types=[
            pltpu.VMEM((_K,), jnp.int32),
            pltpu.VMEM((_K,), jnp.int32),
            pltpu.VMEM((_K, HH), jnp.float32),
            pltpu.VMEM_SHARED((NP, HH), jnp.float32),
            pltpu.SemaphoreType.DMA,
        ],
    )
    def k(xp, src_flat, dst, zrows, out, idx_s, idx_d, rows, agg, sem):
        c = lax.axis_index("c")
        s = lax.axis_index("s")
        pltpu.sync_copy(zrows, agg.at[pl.ds(s * rps, rps)])
        plsc.subcore_barrier()

        def body(i, carry):
            base = s * eper + i * _K
            pltpu.sync_copy(src_flat.at[pl.ds(c * EP + base, _K)], idx_s)
            pltpu.sync_copy(dst.at[pl.ds(base, _K)], idx_d)
            pltpu.async_copy(xp.at[idx_s], rows, sem).wait()
            pltpu.sync_copy(rows, agg.at[idx_d], add=True)
            return carry

        lax.fori_loop(0, steps, body, 0)
        plsc.subcore_barrier()
        pltpu.sync_copy(agg.at[pl.ds(s * rps, rps)],
                        out.at[c, pl.ds(s * rps, rps)])

    return k


def _bn(n):
    return 1000 if n % 1000 == 0 else n


def _dinv_of(deg_ref):
    cnt = deg_ref[0, :, 0:1]
    return lax.rsqrt(cnt + 1.0)


def _mm_split(x, w, deg2=None):
    n, f = x.shape
    h = w.shape[1]
    hh = h // 2
    bn = _bn(n)
    nb = n // bn
    scaled = deg2 is not None

    def body(*refs):
        if scaled:
            x_ref, w_ref, d_ref, o_ref = refs
        else:
            x_ref, w_ref, o_ref = refs
        acc = jnp.dot(x_ref[...], w_ref[...], preferred_element_type=jnp.float32)
        if scaled:
            acc = acc * _dinv_of(d_ref)
        o_ref[0] = acc

    in_specs = [
        pl.BlockSpec((bn, f), lambda i, j: (i, 0)),
        pl.BlockSpec((f, hh), lambda i, j: (0, j)),
    ]
    args = [x, w]
    if scaled:
        in_specs.append(pl.BlockSpec((2, bn, 128), lambda i, j: (0, i, 0)))
        args.append(deg2)
    return pl.pallas_call(
        body,
        grid=(nb, 2),
        in_specs=in_specs,
        out_specs=pl.BlockSpec((1, bn, hh), lambda i, j: (j, i, 0)),
        out_shape=jax.ShapeDtypeStruct((2, n, hh), jnp.float32),
    )(*args)


def _gcn_post(agg2, xs2, deg2, b1):
    _, n, hh = xs2.shape
    h = 2 * hh
    bn = _bn(n)
    nb = n // bn
    b1r = b1.reshape(1, h)

    def body(a_ref, x_ref, d_ref, b_ref, o_ref):
        a = jnp.concatenate([a_ref[0], a_ref[1]], axis=1)
        xs = jnp.concatenate([x_ref[0], x_ref[1]], axis=1)
        dinv = _dinv_of(d_ref)
        o_ref[...] = jnp.maximum(dinv * (a + xs) + b_ref[...], 0.0)

    return pl.pallas_call(
        body,
        grid=(nb,),
        in_specs=[
            pl.BlockSpec((2, bn, hh), lambda i: (0, i, 0)),
            pl.BlockSpec((2, bn, hh), lambda i: (0, i, 0)),
            pl.BlockSpec((2, bn, 128), lambda i: (0, i, 0)),
            pl.BlockSpec((1, h), lambda i: (0, 0)),
        ],
        out_specs=pl.BlockSpec((bn, h), lambda i: (i, 0)),
        out_shape=jax.ShapeDtypeStruct((n, h), jnp.float32),
    )(agg2, xs2, deg2, b1r)


def _gru(agg2, hprev, w_iht, w_hht, b_ih, b_hh, final_relu):
    n, h = hprev.shape
    hh = h // 2
    bn = _bn(n)
    nb = n // bn
    b_ihr = b_ih.reshape(1, 3 * h)
    b_hhr = b_hh.reshape(1, 3 * h)

    def body(a_ref, h_ref, wi_ref, wh_ref, bi_ref, bh_ref, o_ref):
        m = jnp.concatenate([a_ref[0], a_ref[1]], axis=1)
        hp = h_ref[...]
        gi = jnp.dot(m, wi_ref[...], preferred_element_type=jnp.float32) + bi_ref[...]
        gh = jnp.dot(hp, wh_ref[...], preferred_element_type=jnp.float32) + bh_ref[...]
        r = jax.nn.sigmoid(gi[:, :h] + gh[:, :h])
        z = jax.nn.sigmoid(gi[:, h:2 * h] + gh[:, h:2 * h])
        ncand = jnp.tanh(gi[:, 2 * h:] + r * gh[:, 2 * h:])
        out = (1.0 - z) * ncand + z * hp
        if final_relu:
            out = jnp.maximum(out, 0.0)
        o_ref[...] = out

    return pl.pallas_call(
        body,
        grid=(nb,),
        in_specs=[
            pl.BlockSpec((2, bn, hh), lambda i: (0, i, 0)),
            pl.BlockSpec((bn, h), lambda i: (i, 0)),
            pl.BlockSpec((h, 3 * h), lambda i: (0, 0)),
            pl.BlockSpec((h, 3 * h), lambda i: (0, 0)),
            pl.BlockSpec((1, 3 * h), lambda i: (0, 0)),
            pl.BlockSpec((1, 3 * h), lambda i: (0, 0)),
        ],
        out_specs=pl.BlockSpec((bn, h), lambda i: (i, 0)),
        out_shape=jax.ShapeDtypeStruct((n, h), jnp.float32),
    )(agg2, hprev, w_iht, w_hht, b_ihr, b_hhr)


def _ceil_to(a, m):
    return (a + m - 1) // m * m


def kernel(x, edge_index, W1, b1, Wg, W_ih, W_hh, b_ih, b_hh):
    n = x.shape[0]
    e = edge_index.shape[1]
    h = W1.shape[1]
    hh = h // 2
    np_ = _ceil_to(n, 8 * _NS)
    ep = _ceil_to(e, _K * _NS)
    pad_e = ep - e
    src = edge_index[0]
    dst = edge_index[1]
    zpad = jnp.zeros((pad_e,), jnp.int32)
    src_flat = jnp.concatenate([src, zpad, src + n, zpad])
    dst_pad = jnp.concatenate([dst, jnp.full((pad_e,), np_ - 1, jnp.int32)])
    zrows = jnp.zeros((np_ // _NS, hh), jnp.float32)
    edge_k = _edge_pass_kernel(np_, ep, hh)
    ones_tab = jnp.ones((8, hh), jnp.float32)
    zero_idx = jnp.zeros((2 * ep,), jnp.int32)
    deg2 = edge_k(ones_tab, zero_idx, dst_pad, zrows)

    def edge(x2):
        return edge_k(x2.reshape(2 * n, hh), src_flat, dst_pad, zrows)

    xs2 = _mm_split(x, W1, deg2)
    agg2 = edge(xs2)
    hcur = _gcn_post(agg2, xs2, deg2, b1)

    w_iht = W_ih.T
    w_hht = W_hh.T
    nl = Wg.shape[0]
    for i in range(nl):
        m2 = _mm_split(hcur, Wg[i])
        aggm2 = edge(m2)
        hcur = _gru(aggm2, hcur, w_iht, w_hht, b_ih, b_hh, final_relu=(i == nl - 1))
    return hcur
